# baseline (speedup 1.0000x reference)
import functools

import jax
import jax.numpy as jnp
from jax import lax
from jax.experimental import pallas as pl
from jax.experimental.pallas import tpu as pltpu
from jax.experimental.pallas import tpu_sc as plsc

N = 10000
E = 160000
H = 256
T = 32
N_PAD = 10240
M_BLK = 512

NC = 2
NS = 16
NW = NC * NS
RPT = N_PAD // NW
EW = 4000
CWIN = 2048
GCH = 128
GPAD = 2 * GCH
CBUF = CWIN + GPAD + 128
CMAXH = ((E + GPAD + CWIN - 1) // CWIN + 1) * CWIN

_mesh = plsc.VectorSubcoreMesh(
    core_axis_name="c", subcore_axis_name="s", num_cores=NC, num_subcores=NS)


def _prep_body(src_hbm, dst_hbm, dis_out, csrc_out, cdst_out, cnts_out,
               swin, dwin, hist, cwsrc, cwdst, cbuf, dacc):
    c = lax.axis_index("c")
    s = lax.axis_index("s")
    wid = c * NS + s
    lo = wid * RPT

    zero16f = jnp.zeros((16,), jnp.float32)
    zero16i = jnp.zeros((16,), jnp.int32)
    garb16 = jnp.full((16,), RPT, jnp.int32)
    ones16 = jnp.ones((16,), jnp.float32)

    def zh(i, _):
        hist[pl.ds(i * 16, 16)] = zero16f
        return 0
    lax.fori_loop(0, (RPT + 16) // 16, zh, 0)

    def window(w, carry):
        woff = pl.multiple_of(w * EW, 8)
        pltpu.sync_copy(src_hbm.at[pl.ds(woff, EW)], swin)
        pltpu.sync_copy(dst_hbm.at[pl.ds(woff, EW)], dwin)

        def body(i, carry):
            cnt, flushed = carry
            s16 = swin[pl.ds(i * 16, 16)]
            d16 = dwin[pl.ds(i * 16, 16)]
            dloc = d16 - lo
            m = (dloc >= 0) & (dloc < RPT)
            dl2 = jnp.where(m, dloc, RPT)
            plsc.addupdate_scatter(hist, [dl2], ones16, mask=m)
            off = cnt - flushed
            plsc.store_compressed(cwsrc.at[pl.ds(off, 16)], s16, mask=m)
            plsc.store_compressed(cwdst.at[pl.ds(off, 16)], dl2, mask=m)
            cnt = cnt + plsc.all_reduce_population_count(m)[0]
            do_flush = (cnt - flushed) >= CWIN

            @pl.when(do_flush)
            def _():
                fof = pl.multiple_of(flushed, CWIN)
                pltpu.sync_copy(cwsrc.at[pl.ds(0, CWIN)],
                                csrc_out.at[wid, pl.ds(fof, CWIN)])
                pltpu.sync_copy(cwdst.at[pl.ds(0, CWIN)],
                                cdst_out.at[wid, pl.ds(fof, CWIN)])
                ts = cwsrc[pl.ds(CWIN, 16)]
                td = cwdst[pl.ds(CWIN, 16)]
                ts2 = cwsrc[pl.ds(CWIN + 16, 16)]
                td2 = cwdst[pl.ds(CWIN + 16, 16)]
                cwsrc[pl.ds(0, 16)] = ts
                cwdst[pl.ds(0, 16)] = td
                cwsrc[pl.ds(16, 16)] = ts2
                cwdst[pl.ds(16, 16)] = td2
            flushed = jnp.where(do_flush, flushed + CWIN, flushed)
            return (cnt, flushed)
        return lax.fori_loop(0, EW // 16, body, carry)
    cnt, flushed = lax.fori_loop(0, E // EW, window, (0, 0))

    off = cnt - flushed
    for k in range(GPAD // 16):
        cwsrc[pl.ds(off + k * 16, 16)] = zero16i
        cwdst[pl.ds(off + k * 16, 16)] = garb16
    fof = pl.multiple_of(flushed, CWIN)
    pltpu.sync_copy(cwsrc.at[pl.ds(0, CWIN)], csrc_out.at[wid, pl.ds(fof, CWIN)])
    pltpu.sync_copy(cwdst.at[pl.ds(0, CWIN)], cdst_out.at[wid, pl.ds(fof, CWIN)])
    pltpu.sync_copy(cwsrc.at[pl.ds(CWIN, GPAD + 128)],
                    csrc_out.at[wid, pl.ds(fof + CWIN, GPAD + 128)])
    pltpu.sync_copy(cwdst.at[pl.ds(CWIN, GPAD + 128)],
                    cdst_out.at[wid, pl.ds(fof + CWIN, GPAD + 128)])

    lane = lax.iota(jnp.int32, 16)
    cbuf[...] = jnp.where(lane == 0, cnt, 0)
    pltpu.sync_copy(cbuf, cnts_out.at[pl.ds(pl.multiple_of(wid * 16, 16), 16)])

    def nwt(j, _):
        x = hist[pl.ds(j * 16, 16)] + 1.0
        iv = plsc.bitcast(x, jnp.int32)
        iv = 0x5F3759DF - lax.shift_right_logical(iv, 1)
        y = plsc.bitcast(iv, jnp.float32)
        for _ in range(3):
            y = y * (1.5 - 0.5 * x * y * y)
        dacc[pl.ds(j * 16, 16)] = y
        return 0
    lax.fori_loop(0, RPT // 16, nwt, 0)
    pltpu.sync_copy(dacc, dis_out.at[pl.ds(pl.multiple_of(wid * RPT, 8), RPT)])


_prep = pl.kernel(
    _prep_body,
    out_type=(
        jax.ShapeDtypeStruct((N_PAD,), jnp.float32),
        jax.ShapeDtypeStruct((NW, CMAXH), jnp.int32),
        jax.ShapeDtypeStruct((NW, CMAXH), jnp.int32),
        jax.ShapeDtypeStruct((NW * 16,), jnp.int32),
    ),
    mesh=_mesh,
    compiler_params=pltpu.CompilerParams(needs_layout_passes=False),
    scratch_types=[
        pltpu.VMEM((EW,), jnp.int32),
        pltpu.VMEM((EW,), jnp.int32),
        pltpu.VMEM((RPT + 16,), jnp.float32),
        pltpu.VMEM((CBUF,), jnp.int32),
        pltpu.VMEM((CBUF,), jnp.int32),
        pltpu.VMEM((16,), jnp.int32),
        pltpu.VMEM((RPT,), jnp.float32),
    ],
)


def _prop_body(zs_hbm, zpk_hbm, csrc_hbm, cdst_hbm, cnts_hbm, out_hbm,
               csrc, cdst, cbuf, gbuf0, gbuf1, acc, sem0, sem1):
    c = lax.axis_index("c")
    s = lax.axis_index("s")
    wid = c * NS + s
    rof = pl.multiple_of(wid * RPT, 8)
    pltpu.sync_copy(cnts_hbm.at[pl.ds(pl.multiple_of(wid * 16, 16), 16)], cbuf)
    pltpu.sync_copy(zs_hbm.at[pl.ds(rof, RPT)], acc.at[pl.ds(0, RPT)])
    cnt = jnp.sum(cbuf[...])
    nch = lax.div(cnt + (GCH - 1), GCH)
    cpw = CWIN // GCH
    nwin = lax.div(nch + (cpw - 1), cpw)

    hi_mask = jnp.full((16,), -65536, jnp.int32)

    def add_chunk(gbuf, g):
        def grp(jg, _):
            r16 = cdst[pl.ds(g * GCH + jg * 16, 16)]
            for j in range(16):
                r = r16[j]
                row = jg * 16 + j
                for k in range(H // 32):
                    w = gbuf[row, pl.ds(k * 16, 16)]
                    lo = plsc.bitcast(lax.shift_left(w, 16), jnp.float32)
                    hi = plsc.bitcast(w & hi_mask, jnp.float32)
                    plsc.addupdate(acc.at[r, pl.ds(k * 16, 16)], lo)
                    plsc.addupdate(acc.at[r, pl.ds(H // 2 + k * 16, 16)], hi)
            return 0
        lax.fori_loop(0, GCH // 16, grp, 0)

    def gath(gbuf, sem, g):
        return pltpu.async_copy(zpk_hbm.at[csrc.at[pl.ds(g * GCH, GCH)]], gbuf, sem)

    def win(w, _):
        woff = pl.multiple_of(w * CWIN, 128)
        pltpu.sync_copy(csrc_hbm.at[wid, pl.ds(woff, CWIN)], csrc)
        pltpu.sync_copy(cdst_hbm.at[wid, pl.ds(woff, CWIN)], cdst)
        lch = jnp.minimum(nch - w * cpw, cpw)

        gath(gbuf0, sem0, 0)

        def pair(p, _):
            c0 = 2 * p
            c1 = c0 + 1

            @pl.when(c1 < lch)
            def _():
                gath(gbuf1, sem1, c1)
            pltpu.make_async_copy(zpk_hbm.at[pl.ds(0, GCH)], gbuf0, sem0).wait()
            add_chunk(gbuf0, c0)

            @pl.when(c0 + 2 < lch)
            def _():
                gath(gbuf0, sem0, c0 + 2)

            @pl.when(c1 < lch)
            def _():
                pltpu.make_async_copy(zpk_hbm.at[pl.ds(0, GCH)], gbuf1, sem1).wait()
                add_chunk(gbuf1, c1)
            return 0
        lax.fori_loop(0, lax.div(lch + 1, 2), pair, 0)
        return 0
    lax.fori_loop(0, nwin, win, 0)
    pltpu.sync_copy(acc.at[pl.ds(0, RPT)], out_hbm.at[pl.ds(rof, RPT)])


_prop = pl.kernel(
    _prop_body,
    out_type=jax.ShapeDtypeStruct((N_PAD, H), jnp.float32),
    mesh=_mesh,
    compiler_params=pltpu.CompilerParams(needs_layout_passes=False),
    scratch_types=[
        pltpu.VMEM((CWIN,), jnp.int32),
        pltpu.VMEM((CWIN,), jnp.int32),
        pltpu.VMEM((16,), jnp.int32),
        pltpu.VMEM((GCH, H // 2), jnp.int32),
        pltpu.VMEM((GCH, H // 2), jnp.int32),
        pltpu.VMEM((RPT + 8, H), jnp.float32),
        pltpu.SemaphoreType.DMA,
        pltpu.SemaphoreType.DMA,
    ],
)


def _pack_rows(zs):
    a = lax.bitcast_convert_type(zs[:, :H // 2].astype(jnp.bfloat16), jnp.uint16)
    b = lax.bitcast_convert_type(zs[:, H // 2:].astype(jnp.bfloat16), jnp.uint16)
    return (a.astype(jnp.int32) | (b.astype(jnp.int32) << 16)).astype(jnp.int32)


def _embed_matmul_body(hs_ref, ho_ref, t_ref, dis_ref, wt1_ref, bt1_ref,
                       wt2_ref, bt2_ref, wg1_ref, zs_ref, zpk_ref):
    t = t_ref[...]
    h1 = t * wt1_ref[...] + bt1_ref[...]
    h1 = h1 * jax.nn.sigmoid(h1)
    temb = jnp.dot(h1, wt2_ref[...], preferred_element_type=jnp.float32)
    temb = temb + bt2_ref[...]
    z = jnp.dot(hs_ref[...], wg1_ref[0:H, :], preferred_element_type=jnp.float32)
    z += jnp.dot(ho_ref[...], wg1_ref[H:2 * H, :], preferred_element_type=jnp.float32)
    z += jnp.dot(temb, wg1_ref[2 * H:3 * H, :], preferred_element_type=jnp.float32)
    zs = z * dis_ref[...]
    zs_ref[...] = zs
    zpk_ref[...] = _pack_rows(zs)


def _embed_matmul(hs, ho, t2d, dis2d, wt1, bt1, wt2, bt2, wg1):
    grid = (N_PAD // M_BLK,)
    return pl.pallas_call(
        _embed_matmul_body,
        grid=grid,
        in_specs=[
            pl.BlockSpec((M_BLK, H), lambda i: (i, 0)),
            pl.BlockSpec((M_BLK, H), lambda i: (i, 0)),
            pl.BlockSpec((M_BLK, 1), lambda i: (i, 0)),
            pl.BlockSpec((M_BLK, 1), lambda i: (i, 0)),
            pl.BlockSpec((1, T), lambda i: (0, 0)),
            pl.BlockSpec((1, T), lambda i: (0, 0)),
            pl.BlockSpec((T, H), lambda i: (0, 0)),
            pl.BlockSpec((1, H), lambda i: (0, 0)),
            pl.BlockSpec((3 * H, H), lambda i: (0, 0)),
        ],
        out_specs=[pl.BlockSpec((M_BLK, H), lambda i: (i, 0)),
                   pl.BlockSpec((M_BLK, H // 2), lambda i: (i, 0))],
        out_shape=(jax.ShapeDtypeStruct((N_PAD, H), jnp.float32),
                   jax.ShapeDtypeStruct((N_PAD, H // 2), jnp.int32)),
    )(hs, ho, t2d, dis2d, wt1, bt1, wt2, bt2, wg1)


def _relu_matmul_body(s_ref, dis_ref, b_ref, w_ref, zs_ref, zpk_ref):
    h = jnp.maximum(s_ref[...] * dis_ref[...] + b_ref[...], 0.0)
    zs = jnp.dot(h, w_ref[...], preferred_element_type=jnp.float32) * dis_ref[...]
    zs_ref[...] = zs
    zpk_ref[...] = _pack_rows(zs)


def _relu_matmul(s, dis2d, b, w):
    grid = (N_PAD // M_BLK,)
    return pl.pallas_call(
        _relu_matmul_body,
        grid=grid,
        in_specs=[
            pl.BlockSpec((M_BLK, H), lambda i: (i, 0)),
            pl.BlockSpec((M_BLK, 1), lambda i: (i, 0)),
            pl.BlockSpec((1, H), lambda i: (0, 0)),
            pl.BlockSpec((H, H), lambda i: (0, 0)),
        ],
        out_specs=[pl.BlockSpec((M_BLK, H), lambda i: (i, 0)),
                   pl.BlockSpec((M_BLK, H // 2), lambda i: (i, 0))],
        out_shape=(jax.ShapeDtypeStruct((N_PAD, H), jnp.float32),
                   jax.ShapeDtypeStruct((N_PAD, H // 2), jnp.int32)),
    )(s, dis2d, b, w)


def _final_body(s_ref, dis_ref, b_ref, out_ref):
    out_ref[...] = s_ref[...] * dis_ref[...] + b_ref[...]


def _final(s, dis2d, b):
    blk = 400
    grid = (N // blk,)
    return pl.pallas_call(
        _final_body,
        grid=grid,
        in_specs=[
            pl.BlockSpec((blk, H), lambda i: (i, 0)),
            pl.BlockSpec((blk, 1), lambda i: (i, 0)),
            pl.BlockSpec((1, H), lambda i: (0, 0)),
        ],
        out_specs=pl.BlockSpec((blk, H), lambda i: (i, 0)),
        out_shape=jax.ShapeDtypeStruct((N, H), jnp.float32),
    )(s, dis2d, b)


def kernel(h_t_self, h_t_other, t, edge_index, W_t1, b_t1, W_t2, b_t2,
           W_g1, b_g1, W_g2, b_g2, W_g3, b_g3):
    src = edge_index[0]
    dst = edge_index[1]

    dis, csrc, cdst, cnts = _prep(src, dst)
    dis2d = dis.reshape(N_PAD, 1)

    pad = ((0, N_PAD - N), (0, 0))
    hs = jnp.pad(h_t_self, pad)
    ho = jnp.pad(h_t_other, pad)
    t2d = jnp.pad(t.astype(jnp.float32)[:, None], pad)

    zs, zpk = _embed_matmul(hs, ho, t2d, dis2d, W_t1.reshape(1, T),
                            b_t1.reshape(1, T), W_t2, b_t2.reshape(1, H), W_g1)
    s1 = _prop(zs, zpk, csrc, cdst, cnts)
    zs2, zpk2 = _relu_matmul(s1, dis2d, b_g1.reshape(1, H), W_g2)
    s2 = _prop(zs2, zpk2, csrc, cdst, cnts)
    zs3, zpk3 = _relu_matmul(s2, dis2d, b_g2.reshape(1, H), W_g3)
    s3 = _prop(zs3, zpk3, csrc, cdst, cnts)
    return _final(s3, dis2d, b_g3.reshape(1, H))


# speedup vs baseline: 1.1222x; 1.1222x over previous
import functools

import jax
import jax.numpy as jnp
from jax import lax
from jax.experimental import pallas as pl
from jax.experimental.pallas import tpu as pltpu
from jax.experimental.pallas import tpu_sc as plsc

N = 10000
E = 160000
H = 256
T = 32
N_PAD = 10240
M_BLK = 512

NC = 2
NS = 16
NW = NC * NS
RPT = N_PAD // NW
EW = 4000
CWIN = 2048
GCH = 64
GPAD = 2 * GCH
CBUF = CWIN + GPAD + 128
CMAXH = ((E // 2 + GPAD + CWIN - 1) // CWIN + 1) * CWIN

_mesh = plsc.VectorSubcoreMesh(
    core_axis_name="c", subcore_axis_name="s", num_cores=NC, num_subcores=NS)


def _prep_body(src_hbm, dst_hbm, dis_out, csrc_out, cdst_out, cnts_out,
               swin, dwin, hist, cwsA, cwdA, cwsB, cwdB, cbuf, dacc):
    c = lax.axis_index("c")
    s = lax.axis_index("s")
    wid = c * NS + s
    lo = wid * RPT

    zero16f = jnp.zeros((16,), jnp.float32)
    zero16i = jnp.zeros((16,), jnp.int32)
    garb16 = jnp.full((16,), RPT, jnp.int32)
    ones16 = jnp.ones((16,), jnp.float32)

    def zh(i, _):
        hist[pl.ds(i * 16, 16)] = zero16f
        return 0
    lax.fori_loop(0, (RPT + 16) // 16, zh, 0)

    def flush(cws, cwd, wid2, flushed):
        fof = pl.multiple_of(flushed, CWIN)
        pltpu.sync_copy(cws.at[pl.ds(0, CWIN)], csrc_out.at[wid2, pl.ds(fof, CWIN)])
        pltpu.sync_copy(cwd.at[pl.ds(0, CWIN)], cdst_out.at[wid2, pl.ds(fof, CWIN)])
        ts = cws[pl.ds(CWIN, 16)]
        td = cwd[pl.ds(CWIN, 16)]
        ts2 = cws[pl.ds(CWIN + 16, 16)]
        td2 = cwd[pl.ds(CWIN + 16, 16)]
        cws[pl.ds(0, 16)] = ts
        cwd[pl.ds(0, 16)] = td
        cws[pl.ds(16, 16)] = ts2
        cwd[pl.ds(16, 16)] = td2

    def window(w, carry):
        woff = pl.multiple_of(w * EW, 8)
        pltpu.sync_copy(src_hbm.at[pl.ds(woff, EW)], swin)
        pltpu.sync_copy(dst_hbm.at[pl.ds(woff, EW)], dwin)

        def body(i, carry):
            cA, fA, cB, fB = carry
            sA = swin[pl.ds(i * 32, 16)]
            dA = dwin[pl.ds(i * 32, 16)]
            sB = swin[pl.ds(i * 32 + 16, 16)]
            dB = dwin[pl.ds(i * 32 + 16, 16)]
            dlocA = dA - lo
            dlocB = dB - lo
            mA = (dlocA >= 0) & (dlocA < RPT)
            mB = (dlocB >= 0) & (dlocB < RPT)
            dlA = jnp.where(mA, dlocA, RPT)
            dlB = jnp.where(mB, dlocB, RPT)
            plsc.addupdate_scatter(hist, [dlA], ones16, mask=mA)
            plsc.addupdate_scatter(hist, [dlB], ones16, mask=mB)
            plsc.store_compressed(cwsA.at[pl.ds(cA - fA, 16)], sA, mask=mA)
            plsc.store_compressed(cwdA.at[pl.ds(cA - fA, 16)], dlA, mask=mA)
            plsc.store_compressed(cwsB.at[pl.ds(cB - fB, 16)], sB, mask=mB)
            plsc.store_compressed(cwdB.at[pl.ds(cB - fB, 16)], dlB, mask=mB)
            cA = cA + plsc.all_reduce_population_count(mA)[0]
            cB = cB + plsc.all_reduce_population_count(mB)[0]
            flA = (cA - fA) >= CWIN
            flB = (cB - fB) >= CWIN

            @pl.when(flA)
            def _():
                flush(cwsA, cwdA, 2 * wid, fA)

            @pl.when(flB)
            def _():
                flush(cwsB, cwdB, 2 * wid + 1, fB)
            fA = jnp.where(flA, fA + CWIN, fA)
            fB = jnp.where(flB, fB + CWIN, fB)
            return (cA, fA, cB, fB)
        return lax.fori_loop(0, EW // 32, body, carry)
    cA, fA, cB, fB = lax.fori_loop(0, E // EW, window, (0, 0, 0, 0))

    for (cws, cwd, cnt, flushed, wid2) in (
            (cwsA, cwdA, cA, fA, 2 * wid), (cwsB, cwdB, cB, fB, 2 * wid + 1)):
        off = cnt - flushed
        for k in range(GPAD // 16):
            cws[pl.ds(off + k * 16, 16)] = zero16i
            cwd[pl.ds(off + k * 16, 16)] = garb16
        fof = pl.multiple_of(flushed, CWIN)
        pltpu.sync_copy(cws.at[pl.ds(0, CWIN)], csrc_out.at[wid2, pl.ds(fof, CWIN)])
        pltpu.sync_copy(cwd.at[pl.ds(0, CWIN)], cdst_out.at[wid2, pl.ds(fof, CWIN)])
        pltpu.sync_copy(cws.at[pl.ds(CWIN, GPAD + 128)],
                        csrc_out.at[wid2, pl.ds(fof + CWIN, GPAD + 128)])
        pltpu.sync_copy(cwd.at[pl.ds(CWIN, GPAD + 128)],
                        cdst_out.at[wid2, pl.ds(fof + CWIN, GPAD + 128)])

    lane = lax.iota(jnp.int32, 16)
    cbuf[...] = jnp.where(lane == 0, cA, jnp.where(lane == 8, cB, 0))
    pltpu.sync_copy(cbuf, cnts_out.at[pl.ds(pl.multiple_of(wid * 16, 16), 16)])

    def nwt(j, _):
        x = hist[pl.ds(j * 16, 16)] + 1.0
        iv = plsc.bitcast(x, jnp.int32)
        iv = 0x5F3759DF - lax.shift_right_logical(iv, 1)
        y = plsc.bitcast(iv, jnp.float32)
        for _ in range(3):
            y = y * (1.5 - 0.5 * x * y * y)
        dacc[pl.ds(j * 16, 16)] = y
        return 0
    lax.fori_loop(0, RPT // 16, nwt, 0)
    pltpu.sync_copy(dacc, dis_out.at[pl.ds(pl.multiple_of(wid * RPT, 8), RPT)])


_prep = pl.kernel(
    _prep_body,
    out_type=(
        jax.ShapeDtypeStruct((N_PAD,), jnp.float32),
        jax.ShapeDtypeStruct((NW * 2, CMAXH), jnp.int32),
        jax.ShapeDtypeStruct((NW * 2, CMAXH), jnp.int32),
        jax.ShapeDtypeStruct((NW * 16,), jnp.int32),
    ),
    mesh=_mesh,
    compiler_params=pltpu.CompilerParams(needs_layout_passes=False),
    scratch_types=[
        pltpu.VMEM((EW,), jnp.int32),
        pltpu.VMEM((EW,), jnp.int32),
        pltpu.VMEM((RPT + 16,), jnp.float32),
        pltpu.VMEM((CBUF,), jnp.int32),
        pltpu.VMEM((CBUF,), jnp.int32),
        pltpu.VMEM((CBUF,), jnp.int32),
        pltpu.VMEM((CBUF,), jnp.int32),
        pltpu.VMEM((16,), jnp.int32),
        pltpu.VMEM((RPT,), jnp.float32),
    ],
)


def _prop_body(zs_hbm, zpk_hbm, csrc_hbm, cdst_hbm, cnts_hbm, out_hbm,
               csrc, cdst, cbuf, gbuf0, gbuf1, acc, sem0, sem1):
    c = lax.axis_index("c")
    s = lax.axis_index("s")
    wid = c * NS + s
    rof = pl.multiple_of(wid * RPT, 8)
    pltpu.sync_copy(cnts_hbm.at[pl.ds(pl.multiple_of(wid * 16, 16), 16)], cbuf)
    pltpu.sync_copy(zs_hbm.at[pl.ds(rof, RPT)], acc.at[pl.ds(0, RPT)])
    cv = cbuf[...]
    cpw = CWIN // GCH

    hi_mask = jnp.full((16,), -65536, jnp.int32)

    def add_chunk(gbuf, g):
        def grp(jg, _):
            r16 = cdst[pl.ds(g * GCH + jg * 16, 16)]
            for j in range(16):
                r = r16[j]
                row = jg * 16 + j
                for k in range(H // 32):
                    w = gbuf[row, pl.ds(k * 16, 16)]
                    lo = plsc.bitcast(lax.shift_left(w, 16), jnp.float32)
                    hi = plsc.bitcast(w & hi_mask, jnp.float32)
                    plsc.addupdate(acc.at[r, pl.ds(k * 16, 16)], lo)
                    plsc.addupdate(acc.at[r, pl.ds(H // 2 + k * 16, 16)], hi)
            return 0
        lax.fori_loop(0, GCH // 16, grp, 0)

    def gath(gbuf, sem, g):
        return pltpu.async_copy(zpk_hbm.at[csrc.at[pl.ds(g * GCH, GCH)]], gbuf, sem)

    for sub in range(2):
        wid2 = 2 * wid + sub
        cnt = cv[8 * sub]
        nch = lax.div(cnt + (GCH - 1), GCH)
        nwin = lax.div(nch + (cpw - 1), cpw)

        def win(w, _, wid2=wid2, nch=nch):
            woff = pl.multiple_of(w * CWIN, 128)
            pltpu.sync_copy(csrc_hbm.at[wid2, pl.ds(woff, CWIN)], csrc)
            pltpu.sync_copy(cdst_hbm.at[wid2, pl.ds(woff, CWIN)], cdst)
            lch = jnp.minimum(nch - w * cpw, cpw)

            gath(gbuf0, sem0, 0)

            def pair(p, _):
                c0 = 2 * p
                c1 = c0 + 1

                @pl.when(c1 < lch)
                def _():
                    gath(gbuf1, sem1, c1)
                pltpu.make_async_copy(zpk_hbm.at[pl.ds(0, GCH)], gbuf0, sem0).wait()
                add_chunk(gbuf0, c0)

                @pl.when(c0 + 2 < lch)
                def _():
                    gath(gbuf0, sem0, c0 + 2)

                @pl.when(c1 < lch)
                def _():
                    pltpu.make_async_copy(zpk_hbm.at[pl.ds(0, GCH)], gbuf1, sem1).wait()
                    add_chunk(gbuf1, c1)
                return 0
            lax.fori_loop(0, lax.div(lch + 1, 2), pair, 0)
            return 0
        lax.fori_loop(0, nwin, win, 0)
    pltpu.sync_copy(acc.at[pl.ds(0, RPT)], out_hbm.at[pl.ds(rof, RPT)])


_prop = pl.kernel(
    _prop_body,
    out_type=jax.ShapeDtypeStruct((N_PAD, H), jnp.float32),
    mesh=_mesh,
    compiler_params=pltpu.CompilerParams(needs_layout_passes=False),
    scratch_types=[
        pltpu.VMEM((CWIN,), jnp.int32),
        pltpu.VMEM((CWIN,), jnp.int32),
        pltpu.VMEM((16,), jnp.int32),
        pltpu.VMEM((GCH, H // 2), jnp.int32),
        pltpu.VMEM((GCH, H // 2), jnp.int32),
        pltpu.VMEM((RPT + 8, H), jnp.float32),
        pltpu.SemaphoreType.DMA,
        pltpu.SemaphoreType.DMA,
    ],
)


def _pack_rows(zs):
    a = lax.bitcast_convert_type(zs[:, :H // 2].astype(jnp.bfloat16), jnp.uint16)
    b = lax.bitcast_convert_type(zs[:, H // 2:].astype(jnp.bfloat16), jnp.uint16)
    return (a.astype(jnp.int32) | (b.astype(jnp.int32) << 16)).astype(jnp.int32)


def _embed_matmul_body(hs_ref, ho_ref, t_ref, dis_ref, wt1_ref, bt1_ref,
                       wt2_ref, bt2_ref, wg1_ref, zs_ref, zpk_ref):
    t = t_ref[...]
    h1 = t * wt1_ref[...] + bt1_ref[...]
    h1 = h1 * jax.nn.sigmoid(h1)
    temb = jnp.dot(h1, wt2_ref[...], preferred_element_type=jnp.float32)
    temb = temb + bt2_ref[...]
    z = jnp.dot(hs_ref[...], wg1_ref[0:H, :], preferred_element_type=jnp.float32)
    z += jnp.dot(ho_ref[...], wg1_ref[H:2 * H, :], preferred_element_type=jnp.float32)
    z += jnp.dot(temb, wg1_ref[2 * H:3 * H, :], preferred_element_type=jnp.float32)
    zs = z * dis_ref[...]
    zs_ref[...] = zs
    zpk_ref[...] = _pack_rows(zs)


def _embed_matmul(hs, ho, t2d, dis2d, wt1, bt1, wt2, bt2, wg1):
    grid = (N_PAD // M_BLK,)
    return pl.pallas_call(
        _embed_matmul_body,
        grid=grid,
        in_specs=[
            pl.BlockSpec((M_BLK, H), lambda i: (i, 0)),
            pl.BlockSpec((M_BLK, H), lambda i: (i, 0)),
            pl.BlockSpec((M_BLK, 1), lambda i: (i, 0)),
            pl.BlockSpec((M_BLK, 1), lambda i: (i, 0)),
            pl.BlockSpec((1, T), lambda i: (0, 0)),
            pl.BlockSpec((1, T), lambda i: (0, 0)),
            pl.BlockSpec((T, H), lambda i: (0, 0)),
            pl.BlockSpec((1, H), lambda i: (0, 0)),
            pl.BlockSpec((3 * H, H), lambda i: (0, 0)),
        ],
        out_specs=[pl.BlockSpec((M_BLK, H), lambda i: (i, 0)),
                   pl.BlockSpec((M_BLK, H // 2), lambda i: (i, 0))],
        out_shape=(jax.ShapeDtypeStruct((N_PAD, H), jnp.float32),
                   jax.ShapeDtypeStruct((N_PAD, H // 2), jnp.int32)),
    )(hs, ho, t2d, dis2d, wt1, bt1, wt2, bt2, wg1)


def _relu_matmul_body(s_ref, dis_ref, b_ref, w_ref, zs_ref, zpk_ref):
    h = jnp.maximum(s_ref[...] * dis_ref[...] + b_ref[...], 0.0)
    zs = jnp.dot(h, w_ref[...], preferred_element_type=jnp.float32) * dis_ref[...]
    zs_ref[...] = zs
    zpk_ref[...] = _pack_rows(zs)


def _relu_matmul(s, dis2d, b, w):
    grid = (N_PAD // M_BLK,)
    return pl.pallas_call(
        _relu_matmul_body,
        grid=grid,
        in_specs=[
            pl.BlockSpec((M_BLK, H), lambda i: (i, 0)),
            pl.BlockSpec((M_BLK, 1), lambda i: (i, 0)),
            pl.BlockSpec((1, H), lambda i: (0, 0)),
            pl.BlockSpec((H, H), lambda i: (0, 0)),
        ],
        out_specs=[pl.BlockSpec((M_BLK, H), lambda i: (i, 0)),
                   pl.BlockSpec((M_BLK, H // 2), lambda i: (i, 0))],
        out_shape=(jax.ShapeDtypeStruct((N_PAD, H), jnp.float32),
                   jax.ShapeDtypeStruct((N_PAD, H // 2), jnp.int32)),
    )(s, dis2d, b, w)


def _final_body(s_ref, dis_ref, b_ref, out_ref):
    out_ref[...] = s_ref[...] * dis_ref[...] + b_ref[...]


def _final(s, dis2d, b):
    blk = 400
    grid = (N // blk,)
    return pl.pallas_call(
        _final_body,
        grid=grid,
        in_specs=[
            pl.BlockSpec((blk, H), lambda i: (i, 0)),
            pl.BlockSpec((blk, 1), lambda i: (i, 0)),
            pl.BlockSpec((1, H), lambda i: (0, 0)),
        ],
        out_specs=pl.BlockSpec((blk, H), lambda i: (i, 0)),
        out_shape=jax.ShapeDtypeStruct((N, H), jnp.float32),
    )(s, dis2d, b)


def kernel(h_t_self, h_t_other, t, edge_index, W_t1, b_t1, W_t2, b_t2,
           W_g1, b_g1, W_g2, b_g2, W_g3, b_g3):
    src = edge_index[0]
    dst = edge_index[1]

    dis, csrc, cdst, cnts = _prep(src, dst)
    dis2d = dis.reshape(N_PAD, 1)

    pad = ((0, N_PAD - N), (0, 0))
    hs = jnp.pad(h_t_self, pad)
    ho = jnp.pad(h_t_other, pad)
    t2d = jnp.pad(t.astype(jnp.float32)[:, None], pad)

    zs, zpk = _embed_matmul(hs, ho, t2d, dis2d, W_t1.reshape(1, T),
                            b_t1.reshape(1, T), W_t2, b_t2.reshape(1, H), W_g1)
    s1 = _prop(zs, zpk, csrc, cdst, cnts)
    zs2, zpk2 = _relu_matmul(s1, dis2d, b_g1.reshape(1, H), W_g2)
    s2 = _prop(zs2, zpk2, csrc, cdst, cnts)
    zs3, zpk3 = _relu_matmul(s2, dis2d, b_g2.reshape(1, H), W_g3)
    s3 = _prop(zs3, zpk3, csrc, cdst, cnts)
    return _final(s3, dis2d, b_g3.reshape(1, H))


# speedup vs baseline: 1.1843x; 1.0553x over previous
import functools

import jax
import jax.numpy as jnp
from jax import lax
from jax.experimental import pallas as pl
from jax.experimental.pallas import tpu as pltpu
from jax.experimental.pallas import tpu_sc as plsc

N = 10000
E = 160000
H = 256
T = 32
N_PAD = 10240
M_BLK = 512

NC = 2
NS = 16
NW = NC * NS
RPT = N_PAD // NW
EW = 4000
CWIN = 2048
GCH = 64
GPAD = 2 * GCH
CBUF = CWIN + GPAD + 128
CMAXH = ((E // 2 + GPAD + CWIN - 1) // CWIN + 1) * CWIN

_mesh = plsc.VectorSubcoreMesh(
    core_axis_name="c", subcore_axis_name="s", num_cores=NC, num_subcores=NS)


def _prep_body(src_hbm, dst_hbm, dis_out, csrc_out, cdst_out, cnts_out,
               swinA, dwinA, swinB, dwinB, hist, cwsA, cwdA, cwsB, cwdB,
               cbuf, dacc, semA, semB):
    c = lax.axis_index("c")
    s = lax.axis_index("s")
    wid = c * NS + s
    lo = wid * RPT

    zero16f = jnp.zeros((16,), jnp.float32)
    zero16i = jnp.zeros((16,), jnp.int32)
    garb16 = jnp.full((16,), RPT, jnp.int32)
    ones16 = jnp.ones((16,), jnp.float32)

    def zh(i, _):
        hist[pl.ds(i * 16, 16)] = zero16f
        return 0
    lax.fori_loop(0, (RPT + 16) // 16, zh, 0)

    def flush(cws, cwd, wid2, flushed):
        fof = pl.multiple_of(flushed, CWIN)
        pltpu.sync_copy(cws.at[pl.ds(0, CWIN)], csrc_out.at[wid2, pl.ds(fof, CWIN)])
        pltpu.sync_copy(cwd.at[pl.ds(0, CWIN)], cdst_out.at[wid2, pl.ds(fof, CWIN)])
        ts = cws[pl.ds(CWIN, 16)]
        td = cwd[pl.ds(CWIN, 16)]
        ts2 = cws[pl.ds(CWIN + 16, 16)]
        td2 = cwd[pl.ds(CWIN + 16, 16)]
        cws[pl.ds(0, 16)] = ts
        cwd[pl.ds(0, 16)] = td
        cws[pl.ds(16, 16)] = ts2
        cwd[pl.ds(16, 16)] = td2

    def issue(sw, dw, sem, w):
        woff = pl.multiple_of(w * EW, 8)
        pltpu.async_copy(src_hbm.at[pl.ds(woff, EW)], sw, sem)
        pltpu.async_copy(dst_hbm.at[pl.ds(woff, EW)], dw, sem)

    def wait_win(sw, dw, sem):
        pltpu.make_async_copy(src_hbm.at[pl.ds(0, EW)], sw, sem).wait()
        pltpu.make_async_copy(dst_hbm.at[pl.ds(0, EW)], dw, sem).wait()

    def scan_window(swin, dwin, carry):
        def body(i, carry):
            cA, fA, cB, fB = carry
            sA = swin[pl.ds(i * 32, 16)]
            dA = dwin[pl.ds(i * 32, 16)]
            sB = swin[pl.ds(i * 32 + 16, 16)]
            dB = dwin[pl.ds(i * 32 + 16, 16)]
            dlocA = dA - lo
            dlocB = dB - lo
            mA = (dlocA >= 0) & (dlocA < RPT)
            mB = (dlocB >= 0) & (dlocB < RPT)
            dlA = jnp.where(mA, dlocA, RPT)
            dlB = jnp.where(mB, dlocB, RPT)
            plsc.addupdate_scatter(hist, [dlA], ones16, mask=mA)
            plsc.addupdate_scatter(hist, [dlB], ones16, mask=mB)
            plsc.store_compressed(cwsA.at[pl.ds(cA - fA, 16)], sA, mask=mA)
            plsc.store_compressed(cwdA.at[pl.ds(cA - fA, 16)], dlA, mask=mA)
            plsc.store_compressed(cwsB.at[pl.ds(cB - fB, 16)], sB, mask=mB)
            plsc.store_compressed(cwdB.at[pl.ds(cB - fB, 16)], dlB, mask=mB)
            cA = cA + plsc.all_reduce_population_count(mA)[0]
            cB = cB + plsc.all_reduce_population_count(mB)[0]
            flA = (cA - fA) >= CWIN
            flB = (cB - fB) >= CWIN

            @pl.when(flA)
            def _():
                flush(cwsA, cwdA, 2 * wid, fA)

            @pl.when(flB)
            def _():
                flush(cwsB, cwdB, 2 * wid + 1, fB)
            fA = jnp.where(flA, fA + CWIN, fA)
            fB = jnp.where(flB, fB + CWIN, fB)
            return (cA, fA, cB, fB)
        return lax.fori_loop(0, EW // 32, body, carry)

    NWIN_E = E // EW
    issue(swinA, dwinA, semA, 0)

    def wpair(p, carry):
        w0 = 2 * p
        issue(swinB, dwinB, semB, w0 + 1)
        wait_win(swinA, dwinA, semA)
        carry = scan_window(swinA, dwinA, carry)

        @pl.when(w0 + 2 < NWIN_E)
        def _():
            issue(swinA, dwinA, semA, w0 + 2)
        wait_win(swinB, dwinB, semB)
        carry = scan_window(swinB, dwinB, carry)
        return carry
    cA, fA, cB, fB = lax.fori_loop(0, NWIN_E // 2, wpair, (0, 0, 0, 0))

    for (cws, cwd, cnt, flushed, wid2) in (
            (cwsA, cwdA, cA, fA, 2 * wid), (cwsB, cwdB, cB, fB, 2 * wid + 1)):
        off = cnt - flushed
        for k in range(GPAD // 16):
            cws[pl.ds(off + k * 16, 16)] = zero16i
            cwd[pl.ds(off + k * 16, 16)] = garb16
        fof = pl.multiple_of(flushed, CWIN)
        pltpu.sync_copy(cws.at[pl.ds(0, CWIN)], csrc_out.at[wid2, pl.ds(fof, CWIN)])
        pltpu.sync_copy(cwd.at[pl.ds(0, CWIN)], cdst_out.at[wid2, pl.ds(fof, CWIN)])
        pltpu.sync_copy(cws.at[pl.ds(CWIN, GPAD + 128)],
                        csrc_out.at[wid2, pl.ds(fof + CWIN, GPAD + 128)])
        pltpu.sync_copy(cwd.at[pl.ds(CWIN, GPAD + 128)],
                        cdst_out.at[wid2, pl.ds(fof + CWIN, GPAD + 128)])

    lane = lax.iota(jnp.int32, 16)
    cbuf[...] = jnp.where(lane == 0, cA, jnp.where(lane == 8, cB, 0))
    pltpu.sync_copy(cbuf, cnts_out.at[pl.ds(pl.multiple_of(wid * 16, 16), 16)])

    def nwt(j, _):
        x = hist[pl.ds(j * 16, 16)] + 1.0
        iv = plsc.bitcast(x, jnp.int32)
        iv = 0x5F3759DF - lax.shift_right_logical(iv, 1)
        y = plsc.bitcast(iv, jnp.float32)
        for _ in range(3):
            y = y * (1.5 - 0.5 * x * y * y)
        dacc[pl.ds(j * 16, 16)] = y
        return 0
    lax.fori_loop(0, RPT // 16, nwt, 0)
    pltpu.sync_copy(dacc, dis_out.at[pl.ds(pl.multiple_of(wid * RPT, 8), RPT)])


_prep = pl.kernel(
    _prep_body,
    out_type=(
        jax.ShapeDtypeStruct((N_PAD,), jnp.float32),
        jax.ShapeDtypeStruct((NW * 2, CMAXH), jnp.int32),
        jax.ShapeDtypeStruct((NW * 2, CMAXH), jnp.int32),
        jax.ShapeDtypeStruct((NW * 16,), jnp.int32),
    ),
    mesh=_mesh,
    compiler_params=pltpu.CompilerParams(needs_layout_passes=False),
    scratch_types=[
        pltpu.VMEM((EW,), jnp.int32),
        pltpu.VMEM((EW,), jnp.int32),
        pltpu.VMEM((EW,), jnp.int32),
        pltpu.VMEM((EW,), jnp.int32),
        pltpu.VMEM((RPT + 16,), jnp.float32),
        pltpu.VMEM((CBUF,), jnp.int32),
        pltpu.VMEM((CBUF,), jnp.int32),
        pltpu.VMEM((CBUF,), jnp.int32),
        pltpu.VMEM((CBUF,), jnp.int32),
        pltpu.VMEM((16,), jnp.int32),
        pltpu.VMEM((RPT,), jnp.float32),
        pltpu.SemaphoreType.DMA,
        pltpu.SemaphoreType.DMA,
    ],
)


def _prop_body(zs_hbm, zpk_hbm, csrc_hbm, cdst_hbm, cnts_hbm, out_hbm,
               csrc, cdst, cbuf, gbuf0, gbuf1, acc, sem0, sem1):
    c = lax.axis_index("c")
    s = lax.axis_index("s")
    wid = c * NS + s
    rof = pl.multiple_of(wid * RPT, 8)
    pltpu.sync_copy(cnts_hbm.at[pl.ds(pl.multiple_of(wid * 16, 16), 16)], cbuf)
    pltpu.sync_copy(zs_hbm.at[pl.ds(rof, RPT)], acc.at[pl.ds(0, RPT)])
    cv = cbuf[...]
    cpw = CWIN // GCH

    hi_mask = jnp.full((16,), -65536, jnp.int32)

    def add_chunk(gbuf, g):
        def grp(jg, _):
            r16 = cdst[pl.ds(g * GCH + jg * 16, 16)]
            for j in range(16):
                r = r16[j]
                row = jg * 16 + j
                for k in range(H // 32):
                    w = gbuf[row, pl.ds(k * 16, 16)]
                    lo = plsc.bitcast(lax.shift_left(w, 16), jnp.float32)
                    hi = plsc.bitcast(w & hi_mask, jnp.float32)
                    plsc.addupdate(acc.at[r, pl.ds(k * 16, 16)], lo)
                    plsc.addupdate(acc.at[r, pl.ds(H // 2 + k * 16, 16)], hi)
            return 0
        lax.fori_loop(0, GCH // 16, grp, 0)

    def gath(gbuf, sem, g):
        return pltpu.async_copy(zpk_hbm.at[csrc.at[pl.ds(g * GCH, GCH)]], gbuf, sem)

    for sub in range(2):
        wid2 = 2 * wid + sub
        cnt = cv[8 * sub]
        nch = lax.div(cnt + (GCH - 1), GCH)
        nwin = lax.div(nch + (cpw - 1), cpw)

        def win(w, _, wid2=wid2, nch=nch):
            woff = pl.multiple_of(w * CWIN, 128)
            pltpu.sync_copy(csrc_hbm.at[wid2, pl.ds(woff, CWIN)], csrc)
            pltpu.sync_copy(cdst_hbm.at[wid2, pl.ds(woff, CWIN)], cdst)
            lch = jnp.minimum(nch - w * cpw, cpw)

            gath(gbuf0, sem0, 0)

            def pair(p, _):
                c0 = 2 * p
                c1 = c0 + 1

                @pl.when(c1 < lch)
                def _():
                    gath(gbuf1, sem1, c1)
                pltpu.make_async_copy(zpk_hbm.at[pl.ds(0, GCH)], gbuf0, sem0).wait()
                add_chunk(gbuf0, c0)

                @pl.when(c0 + 2 < lch)
                def _():
                    gath(gbuf0, sem0, c0 + 2)

                @pl.when(c1 < lch)
                def _():
                    pltpu.make_async_copy(zpk_hbm.at[pl.ds(0, GCH)], gbuf1, sem1).wait()
                    add_chunk(gbuf1, c1)
                return 0
            lax.fori_loop(0, lax.div(lch + 1, 2), pair, 0)
            return 0
        lax.fori_loop(0, nwin, win, 0)
    pltpu.sync_copy(acc.at[pl.ds(0, RPT)], out_hbm.at[pl.ds(rof, RPT)])


_prop = pl.kernel(
    _prop_body,
    out_type=jax.ShapeDtypeStruct((N_PAD, H), jnp.float32),
    mesh=_mesh,
    compiler_params=pltpu.CompilerParams(needs_layout_passes=False),
    scratch_types=[
        pltpu.VMEM((CWIN,), jnp.int32),
        pltpu.VMEM((CWIN,), jnp.int32),
        pltpu.VMEM((16,), jnp.int32),
        pltpu.VMEM((GCH, H // 2), jnp.int32),
        pltpu.VMEM((GCH, H // 2), jnp.int32),
        pltpu.VMEM((RPT + 8, H), jnp.float32),
        pltpu.SemaphoreType.DMA,
        pltpu.SemaphoreType.DMA,
    ],
)


def _pack_rows(zs):
    a = lax.bitcast_convert_type(zs[:, :H // 2].astype(jnp.bfloat16), jnp.uint16)
    b = lax.bitcast_convert_type(zs[:, H // 2:].astype(jnp.bfloat16), jnp.uint16)
    return (a.astype(jnp.int32) | (b.astype(jnp.int32) << 16)).astype(jnp.int32)


def _embed_matmul_body(hs_ref, ho_ref, t_ref, dis_ref, wt1_ref, bt1_ref,
                       wt2_ref, bt2_ref, wg1_ref, zs_ref, zpk_ref):
    t = t_ref[...]
    h1 = t * wt1_ref[...] + bt1_ref[...]
    h1 = h1 * jax.nn.sigmoid(h1)
    temb = jnp.dot(h1, wt2_ref[...], preferred_element_type=jnp.float32)
    temb = temb + bt2_ref[...]
    z = jnp.dot(hs_ref[...], wg1_ref[0:H, :], preferred_element_type=jnp.float32)
    z += jnp.dot(ho_ref[...], wg1_ref[H:2 * H, :], preferred_element_type=jnp.float32)
    z += jnp.dot(temb, wg1_ref[2 * H:3 * H, :], preferred_element_type=jnp.float32)
    zs = z * dis_ref[...]
    zs_ref[...] = zs
    zpk_ref[...] = _pack_rows(zs)


def _embed_matmul(hs, ho, t2d, dis2d, wt1, bt1, wt2, bt2, wg1):
    grid = (N_PAD // M_BLK,)
    return pl.pallas_call(
        _embed_matmul_body,
        grid=grid,
        in_specs=[
            pl.BlockSpec((M_BLK, H), lambda i: (i, 0)),
            pl.BlockSpec((M_BLK, H), lambda i: (i, 0)),
            pl.BlockSpec((M_BLK, 1), lambda i: (i, 0)),
            pl.BlockSpec((M_BLK, 1), lambda i: (i, 0)),
            pl.BlockSpec((1, T), lambda i: (0, 0)),
            pl.BlockSpec((1, T), lambda i: (0, 0)),
            pl.BlockSpec((T, H), lambda i: (0, 0)),
            pl.BlockSpec((1, H), lambda i: (0, 0)),
            pl.BlockSpec((3 * H, H), lambda i: (0, 0)),
        ],
        out_specs=[pl.BlockSpec((M_BLK, H), lambda i: (i, 0)),
                   pl.BlockSpec((M_BLK, H // 2), lambda i: (i, 0))],
        out_shape=(jax.ShapeDtypeStruct((N_PAD, H), jnp.float32),
                   jax.ShapeDtypeStruct((N_PAD, H // 2), jnp.int32)),
    )(hs, ho, t2d, dis2d, wt1, bt1, wt2, bt2, wg1)


def _relu_matmul_body(s_ref, dis_ref, b_ref, w_ref, zs_ref, zpk_ref):
    h = jnp.maximum(s_ref[...] * dis_ref[...] + b_ref[...], 0.0)
    zs = jnp.dot(h, w_ref[...], preferred_element_type=jnp.float32) * dis_ref[...]
    zs_ref[...] = zs
    zpk_ref[...] = _pack_rows(zs)


def _relu_matmul(s, dis2d, b, w):
    grid = (N_PAD // M_BLK,)
    return pl.pallas_call(
        _relu_matmul_body,
        grid=grid,
        in_specs=[
            pl.BlockSpec((M_BLK, H), lambda i: (i, 0)),
            pl.BlockSpec((M_BLK, 1), lambda i: (i, 0)),
            pl.BlockSpec((1, H), lambda i: (0, 0)),
            pl.BlockSpec((H, H), lambda i: (0, 0)),
        ],
        out_specs=[pl.BlockSpec((M_BLK, H), lambda i: (i, 0)),
                   pl.BlockSpec((M_BLK, H // 2), lambda i: (i, 0))],
        out_shape=(jax.ShapeDtypeStruct((N_PAD, H), jnp.float32),
                   jax.ShapeDtypeStruct((N_PAD, H // 2), jnp.int32)),
    )(s, dis2d, b, w)


def _final_body(s_ref, dis_ref, b_ref, out_ref):
    out_ref[...] = s_ref[...] * dis_ref[...] + b_ref[...]


def _final(s, dis2d, b):
    blk = 400
    grid = (N // blk,)
    return pl.pallas_call(
        _final_body,
        grid=grid,
        in_specs=[
            pl.BlockSpec((blk, H), lambda i: (i, 0)),
            pl.BlockSpec((blk, 1), lambda i: (i, 0)),
            pl.BlockSpec((1, H), lambda i: (0, 0)),
        ],
        out_specs=pl.BlockSpec((blk, H), lambda i: (i, 0)),
        out_shape=jax.ShapeDtypeStruct((N, H), jnp.float32),
    )(s, dis2d, b)


def kernel(h_t_self, h_t_other, t, edge_index, W_t1, b_t1, W_t2, b_t2,
           W_g1, b_g1, W_g2, b_g2, W_g3, b_g3):
    src = edge_index[0]
    dst = edge_index[1]

    dis, csrc, cdst, cnts = _prep(src, dst)
    dis2d = dis.reshape(N_PAD, 1)

    pad = ((0, N_PAD - N), (0, 0))
    hs = jnp.pad(h_t_self, pad)
    ho = jnp.pad(h_t_other, pad)
    t2d = jnp.pad(t.astype(jnp.float32)[:, None], pad)

    zs, zpk = _embed_matmul(hs, ho, t2d, dis2d, W_t1.reshape(1, T),
                            b_t1.reshape(1, T), W_t2, b_t2.reshape(1, H), W_g1)
    s1 = _prop(zs, zpk, csrc, cdst, cnts)
    zs2, zpk2 = _relu_matmul(s1, dis2d, b_g1.reshape(1, H), W_g2)
    s2 = _prop(zs2, zpk2, csrc, cdst, cnts)
    zs3, zpk3 = _relu_matmul(s2, dis2d, b_g2.reshape(1, H), W_g3)
    s3 = _prop(zs3, zpk3, csrc, cdst, cnts)
    return _final(s3, dis2d, b_g3.reshape(1, H))
